# Initial kernel scaffold; baseline (speedup 1.0000x reference)
import functools

import jax
import jax.numpy as jnp
from jax import lax
from jax.experimental import pallas as pl
from jax.experimental.pallas import tpu as pltpu
from jax.experimental.pallas import tpu_sc as plsc

N = 10000
E = 320000
F = 128
NCLASS = 40

NC = 2
NS = 16
NW = NC * NS
CH = 128
NCHUNK = E // CH
RPT = N // NS

_MESH = plsc.VectorSubcoreMesh(core_axis_name="c", subcore_axis_name="s")


def _zero_vmem(ref, nrows, width):
    zv = jnp.zeros((16,), jnp.float32)
    def body(r, _):
        for j in range(width // 16):
            ref[r, pl.ds(j * 16, 16)] = zv
        return ()
    lax.fori_loop(0, nrows, body, ())


DEGW = 16


@functools.partial(
    pl.kernel,
    out_type=jax.ShapeDtypeStruct((2, N, DEGW), jnp.float32),
    mesh=_MESH,
    scratch_types=[
        pltpu.VMEM_SHARED((N, DEGW), jnp.float32),
        pltpu.VMEM((CH,), jnp.int32),
        pltpu.VMEM((CH, DEGW), jnp.float32),
        pltpu.VMEM((RPT, DEGW), jnp.float32),
    ],
)
def _sc_degrees(ei, out, acc, idxv, onev, stage):
    cid = lax.axis_index("c")
    tid = lax.axis_index("s")

    _zero_vmem(stage, RPT, DEGW)
    pltpu.sync_copy(stage, acc.at[pl.ds(tid * RPT, RPT)])

    ones16 = jnp.ones((16,), jnp.float32)
    def fill_ones(r, _):
        onev[r, pl.ds(0, 16)] = ones16
        return ()
    lax.fori_loop(0, CH, fill_ones, ())

    plsc.subcore_barrier()

    nfull = NCHUNK // NS
    def step(i, _):
        c = tid + NS * i
        pltpu.sync_copy(ei.at[cid, pl.ds(c * CH, CH)], idxv)
        pltpu.sync_copy(onev, acc.at[idxv], add=True)
        return ()
    lax.fori_loop(0, nfull, step, ())

    @pl.when(tid < NCHUNK - NS * nfull)
    def _():
        c = NS * nfull + tid
        pltpu.sync_copy(ei.at[cid, pl.ds(c * CH, CH)], idxv)
        pltpu.sync_copy(onev, acc.at[idxv], add=True)

    plsc.subcore_barrier()
    pltpu.sync_copy(acc.at[pl.ds(tid * RPT, RPT)],
                    out.at[cid, pl.ds(tid * RPT, RPT)])


@functools.partial(
    pl.kernel,
    out_type=jax.ShapeDtypeStruct((NC, N, F), jnp.float32),
    mesh=_MESH,
    scratch_types=[
        pltpu.VMEM_SHARED((N, F), jnp.float32),
        pltpu.VMEM((CH,), jnp.int32),
        pltpu.VMEM((CH,), jnp.int32),
        pltpu.VMEM((CH, F), jnp.float32),
        pltpu.VMEM((RPT, F), jnp.float32),
        pltpu.SemaphoreType.DMA,
    ],
)
def _sc_aggregate(xs, ei, out, acc, sidx, didx, rows, stage, sem):
    cid = lax.axis_index("c")
    tid = lax.axis_index("s")
    wid = tid * NC + cid

    _zero_vmem(stage, RPT, F)
    pltpu.sync_copy(stage, acc.at[pl.ds(tid * RPT, RPT)])
    plsc.subcore_barrier()

    nfull = NCHUNK // NW
    def step(i, _):
        c = wid + NW * i
        pltpu.sync_copy(ei.at[0, pl.ds(c * CH, CH)], sidx)
        pltpu.sync_copy(ei.at[1, pl.ds(c * CH, CH)], didx)
        pltpu.async_copy(xs.at[sidx], rows, sem).wait()
        pltpu.sync_copy(rows, acc.at[didx], add=True)
        return ()
    lax.fori_loop(0, nfull, step, ())

    @pl.when(wid < NCHUNK - NW * nfull)
    def _():
        c = NW * nfull + wid
        pltpu.sync_copy(ei.at[0, pl.ds(c * CH, CH)], sidx)
        pltpu.sync_copy(ei.at[1, pl.ds(c * CH, CH)], didx)
        pltpu.async_copy(xs.at[sidx], rows, sem).wait()
        pltpu.sync_copy(rows, acc.at[didx], add=True)

    plsc.subcore_barrier()
    pltpu.sync_copy(acc.at[pl.ds(tid * RPT, RPT)],
                    out.at[cid, pl.ds(tid * RPT, RPT)])


def _norm_from(deg_ref, which):
    d = deg_ref[which, :, 0:1]
    return jnp.where(d > 0.0, lax.rsqrt(jnp.maximum(d, 1.0)), 0.0)


def _tc_prescale_body(x_ref, deg_ref, o_ref):
    o_ref[...] = x_ref[...] * _norm_from(deg_ref, 0)


def _tc_layer1_body(p_ref, deg_ref, w_ref, b_ref, o_ref):
    y = (p_ref[0] + p_ref[1]) * _norm_from(deg_ref, 1)
    h = jnp.dot(y, w_ref[...], preferred_element_type=jnp.float32) + b_ref[...]
    h = jnp.maximum(h, 0.0)
    o_ref[...] = h * _norm_from(deg_ref, 0)


def _tc_layer2_body(p_ref, deg_ref, w_ref, b_ref, wfc_ref, bfc_ref, o_ref):
    y = (p_ref[0] + p_ref[1]) * _norm_from(deg_ref, 1)
    h = jnp.dot(y, w_ref[...], preferred_element_type=jnp.float32) + b_ref[...]
    o_ref[...] = (jnp.dot(h, wfc_ref[...], preferred_element_type=jnp.float32)
                  + bfc_ref[...])


def kernel(x, edge_index, W1, b1, W2, b2, Wfc, bfc):
    ei = edge_index

    degp = _sc_degrees(ei)

    xs1 = pl.pallas_call(
        _tc_prescale_body,
        out_shape=jax.ShapeDtypeStruct((N, F), jnp.float32),
    )(x, degp)

    P1 = _sc_aggregate(xs1, ei)

    xs2 = pl.pallas_call(
        _tc_layer1_body,
        out_shape=jax.ShapeDtypeStruct((N, F), jnp.float32),
    )(P1, degp, W1, b1.reshape(1, F))

    P2 = _sc_aggregate(xs2, ei)

    out = pl.pallas_call(
        _tc_layer2_body,
        out_shape=jax.ShapeDtypeStruct((N, NCLASS), jnp.float32),
    )(P2, degp, W2, b2.reshape(1, F), Wfc, bfc.reshape(1, NCLASS))

    return out


# speedup vs baseline: 4.7597x; 4.7597x over previous
import functools

import jax
import jax.numpy as jnp
from jax import lax
from jax.experimental import pallas as pl
from jax.experimental.pallas import tpu as pltpu
from jax.experimental.pallas import tpu_sc as plsc

N = 10000
E = 320000
F = 128
NCLASS = 40

NC = 2
NS = 16
NW = NC * NS
CH = 128
NCHUNK = E // CH
NPAD = 10240
RPT = NPAD // NS

_MESH = plsc.VectorSubcoreMesh(core_axis_name="c", subcore_axis_name="s")


def _zero_vmem(ref, nrows, width):
    zv = jnp.zeros((16,), jnp.float32)
    def body(r, _):
        for j in range(width // 16):
            ref[r, pl.ds(j * 16, 16)] = zv
        return ()
    lax.fori_loop(0, nrows, body, ())


DEGW = 128


@functools.partial(
    pl.kernel,
    out_type=jax.ShapeDtypeStruct((2, NPAD, DEGW), jnp.float32),
    mesh=_MESH,
    scratch_types=[
        pltpu.VMEM_SHARED((NPAD, DEGW), jnp.float32),
        pltpu.VMEM((CH,), jnp.int32),
        pltpu.VMEM((CH, DEGW), jnp.float32),
    ],
)
def _sc_degrees(eflat, out, acc, idxv, onev):
    cid = lax.axis_index("c")
    tid = lax.axis_index("s")

    _zero_vmem(onev, CH, DEGW)
    for k in range(RPT // CH):
        pltpu.sync_copy(onev, acc.at[pl.ds(tid * RPT + k * CH, CH)])

    ones16 = jnp.ones((16,), jnp.float32)
    def fill_ones(r, _):
        onev[r, pl.ds(0, 16)] = ones16
        return ()
    lax.fori_loop(0, CH, fill_ones, ())

    plsc.subcore_barrier()

    base = cid * E
    nfull = NCHUNK // NS
    def step(i, _):
        c = tid + NS * i
        pltpu.sync_copy(eflat.at[pl.ds(base + c * CH, CH)], idxv)
        pltpu.sync_copy(onev, acc.at[idxv], add=True)
        return ()
    lax.fori_loop(0, nfull, step, ())

    @pl.when(tid < NCHUNK - NS * nfull)
    def _():
        c = NS * nfull + tid
        pltpu.sync_copy(eflat.at[pl.ds(base + c * CH, CH)], idxv)
        pltpu.sync_copy(onev, acc.at[idxv], add=True)

    plsc.subcore_barrier()
    pltpu.sync_copy(acc.at[pl.ds(tid * RPT, RPT)],
                    out.at[cid, pl.ds(tid * RPT, RPT)])


@functools.partial(
    pl.kernel,
    out_type=jax.ShapeDtypeStruct((NC, NPAD, F), jnp.float32),
    mesh=_MESH,
    scratch_types=[
        pltpu.VMEM_SHARED((NPAD, F), jnp.float32),
        pltpu.VMEM((CH,), jnp.int32),
        pltpu.VMEM((CH,), jnp.int32),
        pltpu.VMEM((CH, F), jnp.float32),
        pltpu.SemaphoreType.DMA,
    ],
)
def _sc_aggregate(xs, eflat, out, acc, sidx, didx, rows, sem):
    cid = lax.axis_index("c")
    tid = lax.axis_index("s")
    wid = tid * NC + cid

    _zero_vmem(rows, CH, F)
    for k in range(RPT // CH):
        pltpu.sync_copy(rows, acc.at[pl.ds(tid * RPT + k * CH, CH)])
    plsc.subcore_barrier()

    nfull = NCHUNK // NW
    def step(i, _):
        c = wid + NW * i
        pltpu.sync_copy(eflat.at[pl.ds(c * CH, CH)], sidx)
        pltpu.sync_copy(eflat.at[pl.ds(E + c * CH, CH)], didx)
        pltpu.async_copy(xs.at[sidx], rows, sem).wait()
        pltpu.sync_copy(rows, acc.at[didx], add=True)
        return ()
    lax.fori_loop(0, nfull, step, ())

    @pl.when(wid < NCHUNK - NW * nfull)
    def _():
        c = NW * nfull + wid
        pltpu.sync_copy(eflat.at[pl.ds(c * CH, CH)], sidx)
        pltpu.sync_copy(eflat.at[pl.ds(E + c * CH, CH)], didx)
        pltpu.async_copy(xs.at[sidx], rows, sem).wait()
        pltpu.sync_copy(rows, acc.at[didx], add=True)

    plsc.subcore_barrier()
    pltpu.sync_copy(acc.at[pl.ds(tid * RPT, RPT)],
                    out.at[cid, pl.ds(tid * RPT, RPT)])


def _norm_from(deg_ref, which):
    d = deg_ref[which, :N, 0:1]
    return jnp.where(d > 0.0, lax.rsqrt(jnp.maximum(d, 1.0)), 0.0)


def _tc_prescale_body(x_ref, deg_ref, o_ref):
    o_ref[...] = x_ref[...] * _norm_from(deg_ref, 0)


def _tc_layer1_body(p_ref, deg_ref, w_ref, b_ref, o_ref):
    y = (p_ref[0, :N] + p_ref[1, :N]) * _norm_from(deg_ref, 1)
    h = jnp.dot(y, w_ref[...], preferred_element_type=jnp.float32) + b_ref[...]
    h = jnp.maximum(h, 0.0)
    o_ref[...] = h * _norm_from(deg_ref, 0)


def _tc_layer2_body(p_ref, deg_ref, w_ref, b_ref, wfc_ref, bfc_ref, o_ref):
    y = (p_ref[0, :N] + p_ref[1, :N]) * _norm_from(deg_ref, 1)
    h = jnp.dot(y, w_ref[...], preferred_element_type=jnp.float32) + b_ref[...]
    o_ref[...] = (jnp.dot(h, wfc_ref[...], preferred_element_type=jnp.float32)
                  + bfc_ref[...])


def kernel(x, edge_index, W1, b1, W2, b2, Wfc, bfc):
    eflat = edge_index.reshape(2 * E)

    degp = _sc_degrees(eflat)

    xs1 = pl.pallas_call(
        _tc_prescale_body,
        out_shape=jax.ShapeDtypeStruct((N, F), jnp.float32),
    )(x, degp)

    P1 = _sc_aggregate(xs1, eflat)

    xs2 = pl.pallas_call(
        _tc_layer1_body,
        out_shape=jax.ShapeDtypeStruct((N, F), jnp.float32),
    )(P1, degp, W1, b1.reshape(1, F))

    P2 = _sc_aggregate(xs2, eflat)

    out = pl.pallas_call(
        _tc_layer2_body,
        out_shape=jax.ShapeDtypeStruct((N, NCLASS), jnp.float32),
    )(P2, degp, W2, b2.reshape(1, F), Wfc, bfc.reshape(1, NCLASS))

    return out
